# baseline (speedup 1.0000x reference)
import functools

import jax
import jax.numpy as jnp
from jax import lax
from jax.experimental import pallas as pl
from jax.experimental.pallas import tpu as pltpu
from jax.experimental.pallas import tpu_sc as plsc

B = 4096
MEM_DIM = 500
MSG_DIM = 100
EDGE_DIM = 16
TIME_DIM = 100
RAW_DIM = MEM_DIM + EDGE_DIM + 1 + TIME_DIM
HID = RAW_DIM // 2
G3 = 3 * MEM_DIM

NC, NS = 2, 16
NW = NC * NS
BPW = B // NW


def _sc_mesh():
    return plsc.VectorSubcoreMesh(core_axis_name="c", subcore_axis_name="s",
                                  num_cores=NC, num_subcores=NS)


_SC_PARAMS = pltpu.CompilerParams(use_tc_tiling_on_sc=True)


@functools.lru_cache(maxsize=None)
def _build_sc_gather_mem():
    @functools.partial(
        pl.kernel,
        out_type=(jax.ShapeDtypeStruct((B, 384), jnp.float32),
                  jax.ShapeDtypeStruct((B, 128), jnp.float32),
                  jax.ShapeDtypeStruct((B,), jnp.float32)),
        mesh=_sc_mesh(),
        scratch_types=[
            pltpu.VMEM((BPW,), jnp.int32),
            pltpu.VMEM((BPW, 384), jnp.float32),
            pltpu.VMEM((BPW, 128), jnp.float32),
            pltpu.VMEM((BPW,), jnp.float32),
            pltpu.SemaphoreType.DMA,
            pltpu.SemaphoreType.DMA,
            pltpu.SemaphoreType.DMA,
        ],
        compiler_params=_SC_PARAMS,
    )
    def sc_gather_mem(mem_hbm, tail_hbm, lu_hbm, idx_hbm, rowsa_out,
                      rowsb_out, lu_out, idx_v, rowsa_v, rowsb_v, lu_v,
                      sem1, sem2, sem3):
        wid = lax.axis_index("s") * NC + lax.axis_index("c")
        base = wid * BPW
        pltpu.sync_copy(idx_hbm.at[pl.ds(base, BPW)], idx_v)
        c1 = pltpu.async_copy(mem_hbm.at[idx_v, pl.ds(0, 384)], rowsa_v,
                              sem1)
        c2 = pltpu.async_copy(tail_hbm.at[idx_v], rowsb_v, sem2)
        c3 = pltpu.async_copy(lu_hbm.at[idx_v], lu_v, sem3)
        c1.wait()
        c2.wait()
        c3.wait()
        pltpu.sync_copy(rowsa_v, rowsa_out.at[pl.ds(base, BPW)])
        pltpu.sync_copy(rowsb_v, rowsb_out.at[pl.ds(base, BPW)])
        pltpu.sync_copy(lu_v, lu_out.at[pl.ds(base, BPW)])

    return sc_gather_mem


@functools.lru_cache(maxsize=None)
def _build_sc_gather_hnew():
    @functools.partial(
        pl.kernel,
        out_type=jax.ShapeDtypeStruct((B, MEM_DIM), jnp.float32),
        mesh=_sc_mesh(),
        scratch_types=[
            pltpu.VMEM((BPW,), jnp.int32),
            pltpu.VMEM((BPW, MEM_DIM), jnp.float32),
            pltpu.SemaphoreType.DMA,
        ],
        compiler_params=_SC_PARAMS,
    )
    def sc_gather_hnew(hnew_hbm, idx_hbm, rows_out, idx_v, rows_v, sem1):
        wid = lax.axis_index("s") * NC + lax.axis_index("c")
        base = wid * BPW
        pltpu.sync_copy(idx_hbm.at[pl.ds(base, BPW)], idx_v)
        copies = []
        for g in range(BPW // 16):
            vec = idx_v[pl.ds(g * 16, 16)]
            for k in range(16):
                j = g * 16 + k
                copies.append(pltpu.async_copy(
                    hnew_hbm.at[pl.ds(vec[k], 1)], rows_v.at[pl.ds(j, 1)],
                    sem1))
        for c in copies:
            c.wait()
        pltpu.sync_copy(rows_v, rows_out.at[pl.ds(base, BPW)])

    return sc_gather_hnew


def _sc_gather_mem(mem, tail, lu, idx):
    return _build_sc_gather_mem()(mem, tail, lu, idx)


def _sc_gather_hnew(hnew, idx):
    return _build_sc_gather_hnew()(hnew, idx)


TCOPY_R = 5000


def _tailcopy_body(src_ref, dst_ref):
    dst_ref[:, :116] = src_ref[:, 384:500]
    dst_ref[:, 116:] = jnp.zeros((TCOPY_R, 12), jnp.float32)


def _tail_table(memory):
    n = memory.shape[0]
    return pl.pallas_call(
        _tailcopy_body,
        grid=(n // TCOPY_R,),
        in_specs=[pl.BlockSpec((TCOPY_R, 500), lambda i: (i, 0))],
        out_specs=pl.BlockSpec((TCOPY_R, 128), lambda i: (i, 0)),
        out_shape=jax.ShapeDtypeStruct((n, 128), jnp.float32),
        compiler_params=pltpu.CompilerParams(
            dimension_semantics=("arbitrary",)),
    )(memory)


BB = 512
NBLK = B // BB
JCH = 1024


def _tc_body(hpa, hpb, ef, et, lu, src, tw, tb,
             W1, b1, W2, b2, Wih, Whh, bih, bhh,
             hnew_ref, w_ref):
    f32 = jnp.float32
    M = MEM_DIM
    b = pl.program_id(0)
    dg = functools.partial(lax.dot_general, preferred_element_type=f32)
    hp = jnp.concatenate([hpa[...], hpb[:, :116]], axis=1)
    dt = (et[pl.ds(b * BB, BB)] - lu[pl.ds(b * BB, BB)]).reshape(1, BB)
    twc = tw[...].reshape(TIME_DIM, 1)
    tbc = tb[...].reshape(TIME_DIM, 1)
    teT = jnp.cos(twc * dt + tbc)

    W1all = W1[...]
    acc = dg(hp, W1all[:M], (((1,), (0,)), ((), ())))
    acc = acc + dg(teT, W1all[M + EDGE_DIM + 1:], (((0,), (0,)), ((), ())))
    acc = acc + dg(ef[...], W1all[M:M + EDGE_DIM], (((1,), (0,)), ((), ())))
    acc = acc + dg(dt, W1all[M + EDGE_DIM:M + EDGE_DIM + 1],
                   (((0,), (0,)), ((), ())))
    h1 = jnp.maximum(acc + b1[...].reshape(1, HID), 0.0)
    msg = dg(h1, W2[...], (((1,), (0,)), ((), ()))) \
        + b2[...].reshape(1, MSG_DIM)

    gi = dg(msg, Wih[...], (((1,), (0,)), ((), ()))) \
        + bih[...].reshape(1, G3)
    gh = dg(hp, Whh[...], (((1,), (0,)), ((), ()))) \
        + bhh[...].reshape(1, G3)
    r = jax.nn.sigmoid(gi[:, :M] + gh[:, :M])
    z = jax.nn.sigmoid(gi[:, M:2 * M] + gh[:, M:2 * M])
    n = jnp.tanh(gi[:, 2 * M:] + r * gh[:, 2 * M:])
    hnew_ref[...] = (1.0 - z) * n + z * hp

    si = src[pl.ds(b * BB, BB)].reshape(1, BB)
    sjc = src[...].reshape(1, B)
    sjc = jnp.transpose(sjc, (1, 0))
    w = jnp.full((1, BB), -1, jnp.int32)
    for k in range(B // JCH):
        sj = sjc[k * JCH:(k + 1) * JCH, :]
        jio = lax.broadcasted_iota(jnp.int32, (JCH, BB), 0) + (k * JCH)
        cand = jnp.where(sj == si, jio, -1)
        w = jnp.maximum(w, jnp.max(cand, axis=0, keepdims=True))
    w_ref[...] = w.reshape(BB)


def _tc_main(hpa, hpb, ef, et, lu, src, tw, tb,
             W1, b1, W2, b2, Wih, Whh, bih, bhh):
    def full1(n):
        return pl.BlockSpec((n,), lambda b: (0,))

    def const2(shape):
        return pl.BlockSpec(shape, lambda b: (0, 0))

    in_specs = [
        pl.BlockSpec((BB, 384), lambda b: (b, 0)),
        pl.BlockSpec((BB, 128), lambda b: (b, 0)),
        pl.BlockSpec((BB, EDGE_DIM), lambda b: (b, 0)),
        full1(B),
        full1(B),
        full1(B),
        full1(TIME_DIM),
        full1(TIME_DIM),
        const2((RAW_DIM, HID)),
        full1(HID),
        const2((HID, MSG_DIM)),
        full1(MSG_DIM),
        const2((MSG_DIM, G3)),
        const2((MEM_DIM, G3)),
        full1(G3),
        full1(G3),
    ]
    out_specs = [
        pl.BlockSpec((BB, MEM_DIM), lambda b: (b, 0)),
        pl.BlockSpec((BB,), lambda b: (b,)),
    ]
    out_shape = [
        jax.ShapeDtypeStruct((B, MEM_DIM), jnp.float32),
        jax.ShapeDtypeStruct((B,), jnp.int32),
    ]
    return pl.pallas_call(
        _tc_body,
        grid=(NBLK,),
        in_specs=in_specs,
        out_specs=out_specs,
        out_shape=out_shape,
        compiler_params=pltpu.CompilerParams(
            dimension_semantics=("arbitrary",)),
    )(hpa, hpb, ef, et, lu, src, tw, tb,
      W1, b1, W2, b2, Wih, Whh, bih, bhh)


def kernel(src_nodes, edge_feats, edge_times, memory, last_update,
           time_w, time_b, W1, b1, W2, b2, W_ih, W_hh, b_ih, b_hh):
    src = src_nodes.astype(jnp.int32)
    tail = _tail_table(memory)
    hpa, hpb, lu = _sc_gather_mem(memory, tail, last_update, src)
    h_new, w = _tc_main(hpa, hpb, edge_feats, edge_times, lu, src, time_w,
                        time_b, W1, b1, W2, b2, W_ih, W_hh, b_ih, b_hh)
    return _sc_gather_hnew(h_new, w)


# speedup vs baseline: 1.2868x; 1.2868x over previous
import functools

import jax
import jax.numpy as jnp
from jax import lax
from jax.experimental import pallas as pl
from jax.experimental.pallas import tpu as pltpu
from jax.experimental.pallas import tpu_sc as plsc

B = 4096
MEM_DIM = 500
MSG_DIM = 100
EDGE_DIM = 16
TIME_DIM = 100
RAW_DIM = MEM_DIM + EDGE_DIM + 1 + TIME_DIM
HID = RAW_DIM // 2
G3 = 3 * MEM_DIM

NC, NS = 2, 16
NW = NC * NS
BPW = B // NW


def _sc_mesh():
    return plsc.VectorSubcoreMesh(core_axis_name="c", subcore_axis_name="s",
                                  num_cores=NC, num_subcores=NS)


_SC_PARAMS = pltpu.CompilerParams(use_tc_tiling_on_sc=True)


@functools.lru_cache(maxsize=None)
def _build_sc_gather_mem():
    @functools.partial(
        pl.kernel,
        out_type=(jax.ShapeDtypeStruct((B, 384), jnp.float32),
                  jax.ShapeDtypeStruct((B, 116), jnp.float32),
                  jax.ShapeDtypeStruct((B,), jnp.float32)),
        mesh=_sc_mesh(),
        scratch_types=[
            pltpu.VMEM((BPW,), jnp.int32),
            pltpu.VMEM((BPW, 384), jnp.float32),
            pltpu.VMEM((BPW, 116), jnp.float32),
            pltpu.VMEM((BPW,), jnp.float32),
            pltpu.SemaphoreType.DMA,
            pltpu.SemaphoreType.DMA,
            pltpu.SemaphoreType.DMA,
        ],
        compiler_params=_SC_PARAMS,
    )
    def sc_gather_mem(mem_hbm, lu_hbm, idx_hbm, rowsa_out, rowsb_out,
                      lu_out, idx_v, rowsa_v, rowsb_v, lu_v,
                      sem1, sem2, sem3):
        wid = lax.axis_index("s") * NC + lax.axis_index("c")
        base = wid * BPW
        pltpu.sync_copy(idx_hbm.at[pl.ds(base, BPW)], idx_v)
        c1 = pltpu.async_copy(mem_hbm.at[idx_v, pl.ds(0, 384)], rowsa_v,
                              sem1)
        c3 = pltpu.async_copy(lu_hbm.at[idx_v], lu_v, sem3)
        copies = []
        for g in range(BPW // 16):
            vec = idx_v[pl.ds(g * 16, 16)]
            for k in range(16):
                j = g * 16 + k
                copies.append(pltpu.async_copy(
                    mem_hbm.at[pl.ds(vec[k], 1), pl.ds(384, 116)],
                    rowsb_v.at[pl.ds(j, 1)], sem2))
        c1.wait()
        for c in copies:
            c.wait()
        c3.wait()
        pltpu.sync_copy(rowsa_v, rowsa_out.at[pl.ds(base, BPW)])
        pltpu.sync_copy(rowsb_v, rowsb_out.at[pl.ds(base, BPW)])
        pltpu.sync_copy(lu_v, lu_out.at[pl.ds(base, BPW)])

    return sc_gather_mem


@functools.lru_cache(maxsize=None)
def _build_sc_gather_hnew():
    @functools.partial(
        pl.kernel,
        out_type=jax.ShapeDtypeStruct((B, MEM_DIM), jnp.float32),
        mesh=_sc_mesh(),
        scratch_types=[
            pltpu.VMEM((BPW,), jnp.int32),
            pltpu.VMEM((BPW, MEM_DIM), jnp.float32),
            pltpu.SemaphoreType.DMA,
        ],
        compiler_params=_SC_PARAMS,
    )
    def sc_gather_hnew(hnew_hbm, idx_hbm, rows_out, idx_v, rows_v, sem1):
        wid = lax.axis_index("s") * NC + lax.axis_index("c")
        base = wid * BPW
        pltpu.sync_copy(idx_hbm.at[pl.ds(base, BPW)], idx_v)
        copies = []
        for g in range(BPW // 16):
            vec = idx_v[pl.ds(g * 16, 16)]
            for k in range(16):
                j = g * 16 + k
                copies.append(pltpu.async_copy(
                    hnew_hbm.at[pl.ds(vec[k], 1)], rows_v.at[pl.ds(j, 1)],
                    sem1))
        for c in copies:
            c.wait()
        pltpu.sync_copy(rows_v, rows_out.at[pl.ds(base, BPW)])

    return sc_gather_hnew


def _sc_gather_mem(mem, lu, idx):
    return _build_sc_gather_mem()(mem, lu, idx)


def _sc_gather_hnew(hnew, idx):
    return _build_sc_gather_hnew()(hnew, idx)


BB = 512
NBLK = B // BB
JCH = 1024


def _tc_body(hpa, hpb, ef, et, lu, src, tw, tb,
             W1, b1, W2, b2, Wih, Whh, bih, bhh,
             hnew_ref, w_ref):
    f32 = jnp.float32
    M = MEM_DIM
    b = pl.program_id(0)
    dg = functools.partial(lax.dot_general, preferred_element_type=f32)
    hp = jnp.concatenate([hpa[...], hpb[...]], axis=1)
    dt = (et[pl.ds(b * BB, BB)] - lu[pl.ds(b * BB, BB)]).reshape(1, BB)
    twc = tw[...].reshape(TIME_DIM, 1)
    tbc = tb[...].reshape(TIME_DIM, 1)
    teT = jnp.cos(twc * dt + tbc)

    W1all = W1[...]
    acc = dg(hp, W1all[:M], (((1,), (0,)), ((), ())))
    acc = acc + dg(teT, W1all[M + EDGE_DIM + 1:], (((0,), (0,)), ((), ())))
    acc = acc + dg(ef[...], W1all[M:M + EDGE_DIM], (((1,), (0,)), ((), ())))
    acc = acc + dg(dt, W1all[M + EDGE_DIM:M + EDGE_DIM + 1],
                   (((0,), (0,)), ((), ())))
    h1 = jnp.maximum(acc + b1[...].reshape(1, HID), 0.0)
    msg = dg(h1, W2[...], (((1,), (0,)), ((), ()))) \
        + b2[...].reshape(1, MSG_DIM)

    gi = dg(msg, Wih[...], (((1,), (0,)), ((), ()))) \
        + bih[...].reshape(1, G3)
    gh = dg(hp, Whh[...], (((1,), (0,)), ((), ()))) \
        + bhh[...].reshape(1, G3)
    r = jax.nn.sigmoid(gi[:, :M] + gh[:, :M])
    z = jax.nn.sigmoid(gi[:, M:2 * M] + gh[:, M:2 * M])
    n = jnp.tanh(gi[:, 2 * M:] + r * gh[:, 2 * M:])
    hnew_ref[...] = (1.0 - z) * n + z * hp

    si = src[pl.ds(b * BB, BB)].reshape(1, BB)
    sjc = src[...].reshape(1, B)
    sjc = jnp.transpose(sjc, (1, 0))
    w = jnp.full((1, BB), -1, jnp.int32)
    for k in range(B // JCH):
        sj = sjc[k * JCH:(k + 1) * JCH, :]
        jio = lax.broadcasted_iota(jnp.int32, (JCH, BB), 0) + (k * JCH)
        cand = jnp.where(sj == si, jio, -1)
        w = jnp.maximum(w, jnp.max(cand, axis=0, keepdims=True))
    w_ref[...] = w.reshape(BB)


def _tc_main(hpa, hpb, ef, et, lu, src, tw, tb,
             W1, b1, W2, b2, Wih, Whh, bih, bhh):
    def full1(n):
        return pl.BlockSpec((n,), lambda b: (0,))

    def const2(shape):
        return pl.BlockSpec(shape, lambda b: (0, 0))

    in_specs = [
        pl.BlockSpec((BB, 384), lambda b: (b, 0)),
        pl.BlockSpec((BB, 116), lambda b: (b, 0)),
        pl.BlockSpec((BB, EDGE_DIM), lambda b: (b, 0)),
        full1(B),
        full1(B),
        full1(B),
        full1(TIME_DIM),
        full1(TIME_DIM),
        const2((RAW_DIM, HID)),
        full1(HID),
        const2((HID, MSG_DIM)),
        full1(MSG_DIM),
        const2((MSG_DIM, G3)),
        const2((MEM_DIM, G3)),
        full1(G3),
        full1(G3),
    ]
    out_specs = [
        pl.BlockSpec((BB, MEM_DIM), lambda b: (b, 0)),
        pl.BlockSpec((BB,), lambda b: (b,)),
    ]
    out_shape = [
        jax.ShapeDtypeStruct((B, MEM_DIM), jnp.float32),
        jax.ShapeDtypeStruct((B,), jnp.int32),
    ]
    return pl.pallas_call(
        _tc_body,
        grid=(NBLK,),
        in_specs=in_specs,
        out_specs=out_specs,
        out_shape=out_shape,
        compiler_params=pltpu.CompilerParams(
            dimension_semantics=("arbitrary",)),
    )(hpa, hpb, ef, et, lu, src, tw, tb,
      W1, b1, W2, b2, Wih, Whh, bih, bhh)


def kernel(src_nodes, edge_feats, edge_times, memory, last_update,
           time_w, time_b, W1, b1, W2, b2, W_ih, W_hh, b_ih, b_hh):
    src = src_nodes.astype(jnp.int32)
    hpa, hpb, lu = _sc_gather_mem(memory, last_update, src)
    h_new, w = _tc_main(hpa, hpb, edge_feats, edge_times, lu, src, time_w,
                        time_b, W1, b1, W2, b2, W_ih, W_hh, b_ih, b_hh)
    return _sc_gather_hnew(h_new, w)
